# baseline (speedup 1.0000x reference)
import functools

import numpy as np
import jax
import jax.numpy as jnp
from jax import lax
from jax.experimental import pallas as pl
from jax.experimental.pallas import tpu as pltpu
from jax.experimental.pallas import tpu_sc as plsc

_N_BINS = 15
_BOUNDS = [float(np.float32(b)) for b in np.linspace(0.0, 1.0, _N_BINS + 1)]

_GROUP = 16
_NW = 32
_COLS = 1000
_FULL_CHUNKS = 62
_UF = 8


def _sc_body(logits_hbm, labels_hbm, out_hbm, rows_a, rows_b, labs_a,
             labs_b, val_v, idx_v, z_v, bins_sh, sem_a, sem_b, *, row_off):
    cid = lax.axis_index("c")
    sid = lax.axis_index("s")
    w = sid * 2 + cid
    ngroups_total = (labels_hbm.shape[0] - row_off) // _GROUP
    base_groups = ngroups_total // _NW
    extra = ngroups_total - base_groups * _NW
    ng = base_groups + jnp.where(w < extra, 1, 0)
    ng_max = base_groups + (1 if extra else 0)
    npairs = (ng_max + 1) // 2

    iota = lax.iota(jnp.int32, _GROUP)
    zeros = jnp.zeros((_GROUP,), jnp.float32)
    neg_inf = jnp.full((_GROUP,), -jnp.inf, jnp.float32)

    def _perm(v, sh):
        return lax.gather(
            v, (iota ^ sh)[:, None],
            lax.GatherDimensionNumbers(offset_dims=(),
                                       collapsed_slice_dims=(0,),
                                       start_index_map=(0,)),
            slice_sizes=(1,),
            mode=lax.GatherScatterMode.PROMISE_IN_BOUNDS)

    def _bcast_max(v):
        for sh in (1, 2, 4, 8):
            v = jnp.maximum(v, _perm(v, sh))
        return v

    def _bcast_sum(v):
        for sh in (1, 2, 4, 8):
            v = v + _perm(v, sh)
        return v

    @pl.when(sid == 0)
    def _zero_bins():
        for i in range(3):
            z_v[pl.ds(i * _GROUP, _GROUP)] = zeros
        pltpu.sync_copy(z_v, bins_sh)

    plsc.subcore_barrier()

    def _row0_of(k):
        return row_off + (w + _NW * jnp.minimum(k, ng - 1)) * _GROUP

    def _copies(k, rows_v, labs_v, sem):
        row0 = _row0_of(k)
        return (pltpu.make_async_copy(
                    logits_hbm.at[pl.ds(row0, _GROUP), :], rows_v, sem),
                pltpu.make_async_copy(
                    labels_hbm.at[pl.ds(row0, _GROUP)], labs_v, sem))

    def _issue(k, rows_v, labs_v, sem):
        ca, cb = _copies(k, rows_v, labs_v, sem)
        ca.start()
        cb.start()

    def _wait(k, rows_v, labs_v, sem):
        ca, cb = _copies(k, rows_v, labs_v, sem)
        ca.wait()
        cb.wait()

    def _process(k, rows_v, labs_v):
        valid = k < ng
        m_row = zeros
        s_row = zeros
        a_row = zeros
        labs_vec = labs_v[...]
        loop_chunks = (_FULL_CHUNKS // _UF) * _UF
        for r in range(_GROUP):

            def chunk(ch, ms):
                ms_new = []
                ss_new = []
                for u in range(_UF):
                    x = rows_v[r, pl.ds((ch + u) * _GROUP, _GROUP)]
                    ms_new.append(jnp.maximum(ms[0][u], x))
                    ss_new.append(ms[1][u] + jnp.exp(x))
                return (tuple(ms_new), tuple(ss_new))

            m_cs, s_cs = plsc.parallel_loop(
                0, loop_chunks, _UF,
                carry=(tuple(neg_inf for _ in range(_UF)),
                       tuple(zeros for _ in range(_UF))))(chunk)
            m_cs = list(m_cs)
            s_cs = list(s_cs)
            for ch in range(loop_chunks, _FULL_CHUNKS):
                x = rows_v[r, pl.ds(ch * _GROUP, _GROUP)]
                u = ch - loop_chunks
                m_cs[u] = jnp.maximum(m_cs[u], x)
                s_cs[u] = s_cs[u] + jnp.exp(x)
            xt = rows_v[r, pl.ds(_COLS - _GROUP, _GROUP)]
            tail_new = _COLS - _FULL_CHUNKS * _GROUP
            m_cs[6] = jnp.maximum(m_cs[6], xt)
            s_cs[6] = s_cs[6] + jnp.where(iota >= _GROUP - tail_new,
                                          jnp.exp(xt), 0.0)
            for st in (4, 2, 1):
                for u in range(st):
                    m_cs[u] = jnp.maximum(m_cs[u], m_cs[u + st])
                    s_cs[u] = s_cs[u] + s_cs[u + st]
            lab_r = labs_vec[r]
            xv = rows_v[r, pl.ds((lab_r // _GROUP) * _GROUP, _GROUP)]
            xl_b = _bcast_max(jnp.where(iota == (lab_r % _GROUP), xv,
                                        neg_inf))
            sel = iota == r
            m_row = jnp.where(sel, _bcast_max(m_cs[0]), m_row)
            s_row = jnp.where(sel, _bcast_sum(s_cs[0]), s_row)
            a_row = jnp.where(sel, xl_b, a_row)

        accv = jnp.where(a_row == m_row, 1.0, 0.0).astype(jnp.float32)
        confv = jnp.exp(m_row) / s_row

        bidx = jnp.zeros((_GROUP,), jnp.int32)
        for b in range(1, _N_BINS):
            bidx = bidx + jnp.where(confv > _BOUNDS[b], 1, 0).astype(jnp.int32)
        bidx = jnp.where(valid, bidx, _N_BINS)

        val_v[pl.ds(0, _GROUP)] = jnp.ones((_GROUP,), jnp.float32)
        val_v[pl.ds(_GROUP, _GROUP)] = confv
        val_v[pl.ds(2 * _GROUP, _GROUP)] = accv
        idx_v[pl.ds(0, _GROUP)] = bidx
        idx_v[pl.ds(_GROUP, _GROUP)] = bidx + _GROUP
        idx_v[pl.ds(2 * _GROUP, _GROUP)] = bidx + 2 * _GROUP
        pltpu.sync_copy(val_v, bins_sh.at[idx_v], add=True)

    _issue(0, rows_a, labs_a, sem_a)

    def pair_body(t, carry):
        k0 = 2 * t
        _issue(k0 + 1, rows_b, labs_b, sem_b)
        _wait(k0, rows_a, labs_a, sem_a)
        _process(k0, rows_a, labs_a)
        _issue(k0 + 2, rows_a, labs_a, sem_a)
        _wait(k0 + 1, rows_b, labs_b, sem_b)
        _process(k0 + 1, rows_b, labs_b)
        return carry

    lax.fori_loop(0, npairs, pair_body, 0)
    _wait(2 * npairs, rows_a, labs_a, sem_a)

    plsc.subcore_barrier()

    @pl.when(sid == 0)
    def _writeout():
        pltpu.sync_copy(bins_sh, out_hbm.at[cid])


_B_LO = np.zeros((16,), np.float32)
_B_HI = np.zeros((16,), np.float32)
_B_LO[:_N_BINS] = np.linspace(0.0, 1.0, _N_BINS + 1)[:_N_BINS].astype(np.float32)
_B_HI[:_N_BINS] = np.linspace(0.0, 1.0, _N_BINS + 1)[1:].astype(np.float32)
_B_LO[_N_BINS] = 2.0
_B_HI[_N_BINS] = 3.0


def _tc_body(*refs, nsplit):
    logits_refs = refs[:nsplit]
    labels_ref, bounds_ref, out_ref, acc_ref = refs[nsplit:]
    j = pl.program_id(0)
    nsteps = pl.num_programs(0)

    @pl.when(j == 0)
    def _init():
        acc_ref[...] = jnp.zeros_like(acc_ref)

    lo = bounds_ref[0:1, :]
    hi = bounds_ref[1:2, :]
    br = logits_refs[0].shape[0]

    for k in range(nsplit):
        x = logits_refs[k][...]
        m = jnp.max(x, axis=1, keepdims=True)
        s = jnp.sum(jnp.exp(x - m), axis=1, keepdims=True)
        conf = 1.0 / s

        col = jax.lax.broadcasted_iota(jnp.int32, x.shape, 1)
        amax = jnp.min(jnp.where(x == m, col, x.shape[1]), axis=1,
                       keepdims=True)
        lab = labels_ref[0, k * br:(k + 1) * br]
        acc = (amax == lab).astype(jnp.float32)

        inb = ((conf > lo) & (conf <= hi)).astype(jnp.float32)
        acc_ref[0:1, 0:16] += jnp.sum(inb, axis=0, keepdims=True)
        acc_ref[1:2, 0:16] += jnp.sum(inb * conf, axis=0, keepdims=True)
        acc_ref[2:3, 0:16] += jnp.sum(inb * acc, axis=0, keepdims=True)

    @pl.when(j == nsteps - 1)
    def _fini():
        out_ref[...] = acc_ref[...]


def _tc_bins(logits, labels_i32, n_tc, c):
    nsplit = 4
    block_rows = 1000
    step_rows = nsplit * block_rows
    grid = n_tc // step_rows
    labels3 = labels_i32[:n_tc].reshape(grid, step_rows, 1)
    bounds = jnp.asarray(np.stack([_B_LO, _B_HI]))

    logit_specs = [
        pl.BlockSpec((block_rows, c), lambda i, k=k: (i * nsplit + k, 0))
        for k in range(nsplit)
    ]
    out = pl.pallas_call(
        functools.partial(_tc_body, nsplit=nsplit),
        grid=(grid,),
        in_specs=logit_specs + [
            pl.BlockSpec((1, step_rows, 1), lambda i: (i, 0, 0)),
            pl.BlockSpec((2, 16), lambda i: (0, 0)),
        ],
        out_specs=pl.BlockSpec((8, 128), lambda i: (0, 0)),
        out_shape=jax.ShapeDtypeStruct((8, 128), jnp.float32),
        scratch_shapes=[pltpu.VMEM((8, 128), jnp.float32)],
        compiler_params=pltpu.CompilerParams(
            dimension_semantics=("arbitrary",),
            allow_input_fusion=[True] * 6,
        ),
    )(*([logits] * nsplit), labels3, bounds)
    return out


_N_TC = 44000


def kernel(logits, labels):
    n, c = logits.shape
    labels_i32 = labels.astype(jnp.int32)

    sc_kernel = functools.partial(
        pl.kernel,
        out_type=jax.ShapeDtypeStruct((2, 3 * _GROUP), jnp.float32),
        mesh=plsc.VectorSubcoreMesh(core_axis_name="c", subcore_axis_name="s"),
        compiler_params=pltpu.CompilerParams(use_tc_tiling_on_sc=True),
        scratch_types=[
            pltpu.VMEM((_GROUP, _COLS), jnp.float32),
            pltpu.VMEM((_GROUP, _COLS), jnp.float32),
            pltpu.VMEM((_GROUP,), jnp.int32),
            pltpu.VMEM((_GROUP,), jnp.int32),
            pltpu.VMEM((3 * _GROUP,), jnp.float32),
            pltpu.VMEM((3 * _GROUP,), jnp.int32),
            pltpu.VMEM((3 * _GROUP,), jnp.float32),
            pltpu.VMEM_SHARED((3 * _GROUP,), jnp.float32),
            pltpu.SemaphoreType.DMA,
            pltpu.SemaphoreType.DMA,
        ],
    )(functools.partial(_sc_body, row_off=_N_TC))
    sc_parts = sc_kernel(logits, labels_i32)
    logits_tc = lax.slice(logits, (0, 0), (_N_TC, c))
    tc_parts = _tc_bins(logits_tc, labels_i32, _N_TC, c)

    p = sc_parts[0] + sc_parts[1]
    cnt = p[0:_N_BINS] + tc_parts[0, 0:_N_BINS]
    csum = p[_GROUP:_GROUP + _N_BINS] + tc_parts[1, 0:_N_BINS]
    asum = p[2 * _GROUP:2 * _GROUP + _N_BINS] + tc_parts[2, 0:_N_BINS]
    safe = jnp.maximum(cnt, 1.0)
    contrib = jnp.abs(csum / safe - asum / safe) * (cnt / n)
    ece = jnp.sum(jnp.where(cnt > 0, contrib, 0.0), dtype=jnp.float32)
    return ece.reshape(1)


# speedup vs baseline: 1.1020x; 1.1020x over previous
import functools

import numpy as np
import jax
import jax.numpy as jnp
from jax import lax
from jax.experimental import pallas as pl
from jax.experimental.pallas import tpu as pltpu
from jax.experimental.pallas import tpu_sc as plsc

_N_BINS = 15
_BOUNDS = [float(np.float32(b)) for b in np.linspace(0.0, 1.0, _N_BINS + 1)]

_GROUP = 16
_NW = 32
_COLS = 1000
_FULL_CHUNKS = 62
_UF = 8


def _sc_body(logits_hbm, labels_hbm, out_hbm, rows_a, rows_b, labs_a,
             labs_b, val_v, idx_v, z_v, bins_sh, sem_a, sem_b, *, row_off):
    cid = lax.axis_index("c")
    sid = lax.axis_index("s")
    w = sid * 2 + cid
    ngroups_total = (labels_hbm.shape[0] - row_off) // _GROUP
    base_groups = ngroups_total // _NW
    extra = ngroups_total - base_groups * _NW
    ng = base_groups + jnp.where(w < extra, 1, 0)
    ng_max = base_groups + (1 if extra else 0)
    npairs = (ng_max + 1) // 2

    iota = lax.iota(jnp.int32, _GROUP)
    zeros = jnp.zeros((_GROUP,), jnp.float32)
    neg_inf = jnp.full((_GROUP,), -jnp.inf, jnp.float32)

    def _perm(v, sh):
        return lax.gather(
            v, (iota ^ sh)[:, None],
            lax.GatherDimensionNumbers(offset_dims=(),
                                       collapsed_slice_dims=(0,),
                                       start_index_map=(0,)),
            slice_sizes=(1,),
            mode=lax.GatherScatterMode.PROMISE_IN_BOUNDS)

    def _bcast_max(v):
        for sh in (1, 2, 4, 8):
            v = jnp.maximum(v, _perm(v, sh))
        return v

    def _bcast_sum(v):
        for sh in (1, 2, 4, 8):
            v = v + _perm(v, sh)
        return v

    @pl.when(sid == 0)
    def _zero_bins():
        for i in range(3):
            z_v[pl.ds(i * _GROUP, _GROUP)] = zeros
        pltpu.sync_copy(z_v, bins_sh)

    plsc.subcore_barrier()

    def _row0_of(k):
        return row_off + (w + _NW * jnp.minimum(k, ng - 1)) * _GROUP

    def _copies(k, rows_v, labs_v, sem):
        row0 = _row0_of(k)
        return (pltpu.make_async_copy(
                    logits_hbm.at[pl.ds(row0, _GROUP), :], rows_v, sem),
                pltpu.make_async_copy(
                    labels_hbm.at[pl.ds(row0, _GROUP)], labs_v, sem))

    def _issue(k, rows_v, labs_v, sem):
        ca, cb = _copies(k, rows_v, labs_v, sem)
        ca.start()
        cb.start()

    def _wait(k, rows_v, labs_v, sem):
        ca, cb = _copies(k, rows_v, labs_v, sem)
        ca.wait()
        cb.wait()

    def _process(k, rows_v, labs_v):
        valid = k < ng
        m_row = zeros
        s_row = zeros
        a_row = zeros
        labs_vec = labs_v[...]
        loop_chunks = (_FULL_CHUNKS // _UF) * _UF
        for r in range(_GROUP):

            def chunk(ch, ms):
                ms_new = []
                ss_new = []
                for u in range(_UF):
                    x = rows_v[r, pl.ds((ch + u) * _GROUP, _GROUP)]
                    ms_new.append(jnp.maximum(ms[0][u], x))
                    ss_new.append(ms[1][u] + jnp.exp(x))
                return (tuple(ms_new), tuple(ss_new))

            m_cs, s_cs = plsc.parallel_loop(
                0, loop_chunks, _UF,
                carry=(tuple(neg_inf for _ in range(_UF)),
                       tuple(zeros for _ in range(_UF))))(chunk)
            m_cs = list(m_cs)
            s_cs = list(s_cs)
            for ch in range(loop_chunks, _FULL_CHUNKS):
                x = rows_v[r, pl.ds(ch * _GROUP, _GROUP)]
                u = ch - loop_chunks
                m_cs[u] = jnp.maximum(m_cs[u], x)
                s_cs[u] = s_cs[u] + jnp.exp(x)
            xt = rows_v[r, pl.ds(_COLS - _GROUP, _GROUP)]
            tail_new = _COLS - _FULL_CHUNKS * _GROUP
            m_cs[6] = jnp.maximum(m_cs[6], xt)
            s_cs[6] = s_cs[6] + jnp.where(iota >= _GROUP - tail_new,
                                          jnp.exp(xt), 0.0)
            for st in (4, 2, 1):
                for u in range(st):
                    m_cs[u] = jnp.maximum(m_cs[u], m_cs[u + st])
                    s_cs[u] = s_cs[u] + s_cs[u + st]
            lab_r = labs_vec[r]
            xv = rows_v[r, pl.ds((lab_r // _GROUP) * _GROUP, _GROUP)]
            xl_b = _bcast_max(jnp.where(iota == (lab_r % _GROUP), xv,
                                        neg_inf))
            sel = iota == r
            m_row = jnp.where(sel, _bcast_max(m_cs[0]), m_row)
            s_row = jnp.where(sel, _bcast_sum(s_cs[0]), s_row)
            a_row = jnp.where(sel, xl_b, a_row)

        accv = jnp.where(a_row == m_row, 1.0, 0.0).astype(jnp.float32)
        confv = jnp.exp(m_row) / s_row

        bidx = jnp.zeros((_GROUP,), jnp.int32)
        for b in range(1, _N_BINS):
            bidx = bidx + jnp.where(confv > _BOUNDS[b], 1, 0).astype(jnp.int32)
        bidx = jnp.where(valid, bidx, _N_BINS)

        val_v[pl.ds(0, _GROUP)] = jnp.ones((_GROUP,), jnp.float32)
        val_v[pl.ds(_GROUP, _GROUP)] = confv
        val_v[pl.ds(2 * _GROUP, _GROUP)] = accv
        idx_v[pl.ds(0, _GROUP)] = bidx
        idx_v[pl.ds(_GROUP, _GROUP)] = bidx + _GROUP
        idx_v[pl.ds(2 * _GROUP, _GROUP)] = bidx + 2 * _GROUP
        pltpu.sync_copy(val_v, bins_sh.at[idx_v], add=True)

    _issue(0, rows_a, labs_a, sem_a)

    def pair_body(t, carry):
        k0 = 2 * t
        _issue(k0 + 1, rows_b, labs_b, sem_b)
        _wait(k0, rows_a, labs_a, sem_a)
        _process(k0, rows_a, labs_a)
        _issue(k0 + 2, rows_a, labs_a, sem_a)
        _wait(k0 + 1, rows_b, labs_b, sem_b)
        _process(k0 + 1, rows_b, labs_b)
        return carry

    lax.fori_loop(0, npairs, pair_body, 0)
    _wait(2 * npairs, rows_a, labs_a, sem_a)

    plsc.subcore_barrier()

    @pl.when(sid == 0)
    def _writeout():
        pltpu.sync_copy(bins_sh, out_hbm.at[cid])


_B_LO = np.zeros((16,), np.float32)
_B_HI = np.zeros((16,), np.float32)
_B_LO[:_N_BINS] = np.linspace(0.0, 1.0, _N_BINS + 1)[:_N_BINS].astype(np.float32)
_B_HI[:_N_BINS] = np.linspace(0.0, 1.0, _N_BINS + 1)[1:].astype(np.float32)
_B_LO[_N_BINS] = 2.0
_B_HI[_N_BINS] = 3.0


def _tc_body(*refs, nsplit):
    logits_refs = refs[:nsplit]
    labels_ref, bounds_ref, out_ref, acc_ref = refs[nsplit:]
    j = pl.program_id(0)
    nsteps = pl.num_programs(0)

    @pl.when(j == 0)
    def _init():
        acc_ref[...] = jnp.zeros_like(acc_ref)

    lo = bounds_ref[0:1, :]
    hi = bounds_ref[1:2, :]
    br = logits_refs[0].shape[0]

    for k in range(nsplit):
        x = logits_refs[k][...]
        m = jnp.max(x, axis=1, keepdims=True)
        s = jnp.sum(jnp.exp(x - m), axis=1, keepdims=True)
        conf = 1.0 / s

        col = jax.lax.broadcasted_iota(jnp.int32, x.shape, 1)
        amax = jnp.min(jnp.where(x == m, col, x.shape[1]), axis=1,
                       keepdims=True)
        lab = labels_ref[0, k * br:(k + 1) * br]
        acc = (amax == lab).astype(jnp.float32)

        inb = ((conf > lo) & (conf <= hi)).astype(jnp.float32)
        acc_ref[0:1, 0:16] += jnp.sum(inb, axis=0, keepdims=True)
        acc_ref[1:2, 0:16] += jnp.sum(inb * conf, axis=0, keepdims=True)
        acc_ref[2:3, 0:16] += jnp.sum(inb * acc, axis=0, keepdims=True)

    @pl.when(j == nsteps - 1)
    def _fini():
        out_ref[...] = acc_ref[...]


def _tc_bins(logits, labels_i32, n_tc, c):
    nsplit = 4
    block_rows = 1000
    step_rows = nsplit * block_rows
    grid = n_tc // step_rows
    labels3 = labels_i32[:n_tc].reshape(grid, step_rows, 1)
    bounds = jnp.asarray(np.stack([_B_LO, _B_HI]))

    logit_specs = [
        pl.BlockSpec((block_rows, c), lambda i, k=k: (i * nsplit + k, 0))
        for k in range(nsplit)
    ]
    out = pl.pallas_call(
        functools.partial(_tc_body, nsplit=nsplit),
        grid=(grid,),
        in_specs=logit_specs + [
            pl.BlockSpec((1, step_rows, 1), lambda i: (i, 0, 0)),
            pl.BlockSpec((2, 16), lambda i: (0, 0)),
        ],
        out_specs=pl.BlockSpec((8, 128), lambda i: (0, 0)),
        out_shape=jax.ShapeDtypeStruct((8, 128), jnp.float32),
        scratch_shapes=[pltpu.VMEM((8, 128), jnp.float32)],
        compiler_params=pltpu.CompilerParams(
            dimension_semantics=("arbitrary",),
            allow_input_fusion=[True] * 6,
        ),
    )(*([logits] * nsplit), labels3, bounds)
    return out


_N_TC = 60000


def kernel(logits, labels):
    n, c = logits.shape
    labels_i32 = labels.astype(jnp.int32)

    sc_kernel = functools.partial(
        pl.kernel,
        out_type=jax.ShapeDtypeStruct((2, 3 * _GROUP), jnp.float32),
        mesh=plsc.VectorSubcoreMesh(core_axis_name="c", subcore_axis_name="s"),
        compiler_params=pltpu.CompilerParams(use_tc_tiling_on_sc=True),
        scratch_types=[
            pltpu.VMEM((_GROUP, _COLS), jnp.float32),
            pltpu.VMEM((_GROUP, _COLS), jnp.float32),
            pltpu.VMEM((_GROUP,), jnp.int32),
            pltpu.VMEM((_GROUP,), jnp.int32),
            pltpu.VMEM((3 * _GROUP,), jnp.float32),
            pltpu.VMEM((3 * _GROUP,), jnp.int32),
            pltpu.VMEM((3 * _GROUP,), jnp.float32),
            pltpu.VMEM_SHARED((3 * _GROUP,), jnp.float32),
            pltpu.SemaphoreType.DMA,
            pltpu.SemaphoreType.DMA,
        ],
    )(functools.partial(_sc_body, row_off=_N_TC))
    sc_parts = sc_kernel(logits, labels_i32)
    tc_parts = _tc_bins(logits, labels_i32, _N_TC, c)

    p = sc_parts[0] + sc_parts[1]
    cnt = p[0:_N_BINS] + tc_parts[0, 0:_N_BINS]
    csum = p[_GROUP:_GROUP + _N_BINS] + tc_parts[1, 0:_N_BINS]
    asum = p[2 * _GROUP:2 * _GROUP + _N_BINS] + tc_parts[2, 0:_N_BINS]
    safe = jnp.maximum(cnt, 1.0)
    contrib = jnp.abs(csum / safe - asum / safe) * (cnt / n)
    ece = jnp.sum(jnp.where(cnt > 0, contrib, 0.0), dtype=jnp.float32)
    return ece.reshape(1)


# speedup vs baseline: 1.1236x; 1.0196x over previous
import functools

import numpy as np
import jax
import jax.numpy as jnp
from jax import lax
from jax.experimental import pallas as pl
from jax.experimental.pallas import tpu as pltpu
from jax.experimental.pallas import tpu_sc as plsc

_N_BINS = 15
_BOUNDS = [float(np.float32(b)) for b in np.linspace(0.0, 1.0, _N_BINS + 1)]

_GROUP = 16
_NW = 32
_COLS = 1000
_FULL_CHUNKS = 62
_UF = 8


def _sc_body(logits_hbm, labels_hbm, out_hbm, rows_a, rows_b, labs_a,
             labs_b, val_v, idx_v, z_v, bins_sh, sem_a, sem_b, *, row_off):
    cid = lax.axis_index("c")
    sid = lax.axis_index("s")
    w = sid * 2 + cid
    ngroups_total = (labels_hbm.shape[0] - row_off) // _GROUP
    base_groups = ngroups_total // _NW
    extra = ngroups_total - base_groups * _NW
    ng = base_groups + jnp.where(w < extra, 1, 0)
    ng_max = base_groups + (1 if extra else 0)
    npairs = (ng_max + 1) // 2

    iota = lax.iota(jnp.int32, _GROUP)
    zeros = jnp.zeros((_GROUP,), jnp.float32)
    neg_inf = jnp.full((_GROUP,), -jnp.inf, jnp.float32)

    def _perm(v, sh):
        return lax.gather(
            v, (iota ^ sh)[:, None],
            lax.GatherDimensionNumbers(offset_dims=(),
                                       collapsed_slice_dims=(0,),
                                       start_index_map=(0,)),
            slice_sizes=(1,),
            mode=lax.GatherScatterMode.PROMISE_IN_BOUNDS)

    def _bcast_max(v):
        for sh in (1, 2, 4, 8):
            v = jnp.maximum(v, _perm(v, sh))
        return v

    def _bcast_sum(v):
        for sh in (1, 2, 4, 8):
            v = v + _perm(v, sh)
        return v

    @pl.when(sid == 0)
    def _zero_bins():
        for i in range(3):
            z_v[pl.ds(i * _GROUP, _GROUP)] = zeros
        pltpu.sync_copy(z_v, bins_sh)

    plsc.subcore_barrier()

    def _row0_of(k):
        return row_off + (w + _NW * jnp.minimum(k, ng - 1)) * _GROUP

    def _copies(k, rows_v, labs_v, sem):
        row0 = _row0_of(k)
        return (pltpu.make_async_copy(
                    logits_hbm.at[pl.ds(row0, _GROUP), :], rows_v, sem),
                pltpu.make_async_copy(
                    labels_hbm.at[pl.ds(row0, _GROUP)], labs_v, sem))

    def _issue(k, rows_v, labs_v, sem):
        ca, cb = _copies(k, rows_v, labs_v, sem)
        ca.start()
        cb.start()

    def _wait(k, rows_v, labs_v, sem):
        ca, cb = _copies(k, rows_v, labs_v, sem)
        ca.wait()
        cb.wait()

    def _process(k, rows_v, labs_v):
        valid = k < ng
        m_row = zeros
        s_row = zeros
        a_row = zeros
        labs_vec = labs_v[...]
        loop_chunks = (_FULL_CHUNKS // _UF) * _UF
        for r in range(_GROUP):

            def chunk(ch, ms):
                ms_new = []
                ss_new = []
                for u in range(_UF):
                    x = rows_v[r, pl.ds((ch + u) * _GROUP, _GROUP)]
                    ms_new.append(jnp.maximum(ms[0][u], x))
                    ss_new.append(ms[1][u] + jnp.exp(x))
                return (tuple(ms_new), tuple(ss_new))

            m_cs, s_cs = plsc.parallel_loop(
                0, loop_chunks, _UF,
                carry=(tuple(neg_inf for _ in range(_UF)),
                       tuple(zeros for _ in range(_UF))))(chunk)
            m_cs = list(m_cs)
            s_cs = list(s_cs)
            for ch in range(loop_chunks, _FULL_CHUNKS):
                x = rows_v[r, pl.ds(ch * _GROUP, _GROUP)]
                u = ch - loop_chunks
                m_cs[u] = jnp.maximum(m_cs[u], x)
                s_cs[u] = s_cs[u] + jnp.exp(x)
            xt = rows_v[r, pl.ds(_COLS - _GROUP, _GROUP)]
            tail_new = _COLS - _FULL_CHUNKS * _GROUP
            m_cs[6] = jnp.maximum(m_cs[6], xt)
            s_cs[6] = s_cs[6] + jnp.where(iota >= _GROUP - tail_new,
                                          jnp.exp(xt), 0.0)
            for st in (4, 2, 1):
                for u in range(st):
                    m_cs[u] = jnp.maximum(m_cs[u], m_cs[u + st])
                    s_cs[u] = s_cs[u] + s_cs[u + st]
            lab_r = labs_vec[r]
            xv = rows_v[r, pl.ds((lab_r // _GROUP) * _GROUP, _GROUP)]
            xl_b = _bcast_max(jnp.where(iota == (lab_r % _GROUP), xv,
                                        neg_inf))
            sel = iota == r
            m_row = jnp.where(sel, _bcast_max(m_cs[0]), m_row)
            s_row = jnp.where(sel, _bcast_sum(s_cs[0]), s_row)
            a_row = jnp.where(sel, xl_b, a_row)

        accv = jnp.where(a_row == m_row, 1.0, 0.0).astype(jnp.float32)
        confv = jnp.exp(m_row) / s_row

        bidx = jnp.zeros((_GROUP,), jnp.int32)
        for b in range(1, _N_BINS):
            bidx = bidx + jnp.where(confv > _BOUNDS[b], 1, 0).astype(jnp.int32)
        bidx = jnp.where(valid, bidx, _N_BINS)

        val_v[pl.ds(0, _GROUP)] = jnp.ones((_GROUP,), jnp.float32)
        val_v[pl.ds(_GROUP, _GROUP)] = confv
        val_v[pl.ds(2 * _GROUP, _GROUP)] = accv
        idx_v[pl.ds(0, _GROUP)] = bidx
        idx_v[pl.ds(_GROUP, _GROUP)] = bidx + _GROUP
        idx_v[pl.ds(2 * _GROUP, _GROUP)] = bidx + 2 * _GROUP
        pltpu.sync_copy(val_v, bins_sh.at[idx_v], add=True)

    _issue(0, rows_a, labs_a, sem_a)

    def pair_body(t, carry):
        k0 = 2 * t
        _issue(k0 + 1, rows_b, labs_b, sem_b)
        _wait(k0, rows_a, labs_a, sem_a)
        _process(k0, rows_a, labs_a)
        _issue(k0 + 2, rows_a, labs_a, sem_a)
        _wait(k0 + 1, rows_b, labs_b, sem_b)
        _process(k0 + 1, rows_b, labs_b)
        return carry

    lax.fori_loop(0, npairs, pair_body, 0)
    _wait(2 * npairs, rows_a, labs_a, sem_a)

    plsc.subcore_barrier()

    @pl.when(sid == 0)
    def _writeout():
        pltpu.sync_copy(bins_sh, out_hbm.at[cid])


_B_LO = np.zeros((16,), np.float32)
_B_HI = np.zeros((16,), np.float32)
_B_LO[:_N_BINS] = np.linspace(0.0, 1.0, _N_BINS + 1)[:_N_BINS].astype(np.float32)
_B_HI[:_N_BINS] = np.linspace(0.0, 1.0, _N_BINS + 1)[1:].astype(np.float32)
_B_LO[_N_BINS] = 2.0
_B_HI[_N_BINS] = 3.0


def _tc_body(*refs, nsplit):
    logits_refs = refs[:nsplit]
    labels_ref, bounds_ref, out_ref, acc_ref = refs[nsplit:]
    j = pl.program_id(0)
    nsteps = pl.num_programs(0)

    @pl.when(j == 0)
    def _init():
        acc_ref[...] = jnp.zeros_like(acc_ref)

    lo = bounds_ref[0:1, :]
    hi = bounds_ref[1:2, :]
    br = logits_refs[0].shape[0]

    for k in range(nsplit):
        x = logits_refs[k][...]
        m = jnp.max(x, axis=1, keepdims=True)
        s = jnp.sum(jnp.exp(x - m), axis=1, keepdims=True)
        conf = 1.0 / s

        col = jax.lax.broadcasted_iota(jnp.int32, x.shape, 1)
        amax = jnp.min(jnp.where(x == m, col, x.shape[1]), axis=1,
                       keepdims=True)
        lab = labels_ref[0, k * br:(k + 1) * br]
        acc = (amax == lab).astype(jnp.float32)

        inb = ((conf > lo) & (conf <= hi)).astype(jnp.float32)
        acc_ref[0:1, 0:16] += jnp.sum(inb, axis=0, keepdims=True)
        acc_ref[1:2, 0:16] += jnp.sum(inb * conf, axis=0, keepdims=True)
        acc_ref[2:3, 0:16] += jnp.sum(inb * acc, axis=0, keepdims=True)

    @pl.when(j == nsteps - 1)
    def _fini():
        out_ref[...] = acc_ref[...]


def _tc_bins(logits, labels_i32, n_tc, c):
    nsplit = 4
    block_rows = 1000
    step_rows = nsplit * block_rows
    grid = n_tc // step_rows
    labels3 = labels_i32[:n_tc].reshape(grid, step_rows, 1)
    bounds = jnp.asarray(np.stack([_B_LO, _B_HI]))

    logit_specs = [
        pl.BlockSpec((block_rows, c), lambda i, k=k: (i * nsplit + k, 0))
        for k in range(nsplit)
    ]
    out = pl.pallas_call(
        functools.partial(_tc_body, nsplit=nsplit),
        grid=(grid,),
        in_specs=logit_specs + [
            pl.BlockSpec((1, step_rows, 1), lambda i: (i, 0, 0)),
            pl.BlockSpec((2, 16), lambda i: (0, 0)),
        ],
        out_specs=pl.BlockSpec((8, 128), lambda i: (0, 0)),
        out_shape=jax.ShapeDtypeStruct((8, 128), jnp.float32),
        scratch_shapes=[pltpu.VMEM((8, 128), jnp.float32)],
        compiler_params=pltpu.CompilerParams(
            dimension_semantics=("arbitrary",),
            allow_input_fusion=[True] * 6,
        ),
    )(*([logits] * nsplit), labels3, bounds)
    return out


_N_TC = 64000


def kernel(logits, labels):
    n, c = logits.shape
    labels_i32 = labels.astype(jnp.int32)

    sc_kernel = functools.partial(
        pl.kernel,
        out_type=jax.ShapeDtypeStruct((2, 3 * _GROUP), jnp.float32),
        mesh=plsc.VectorSubcoreMesh(core_axis_name="c", subcore_axis_name="s"),
        compiler_params=pltpu.CompilerParams(use_tc_tiling_on_sc=True),
        scratch_types=[
            pltpu.VMEM((_GROUP, _COLS), jnp.float32),
            pltpu.VMEM((_GROUP, _COLS), jnp.float32),
            pltpu.VMEM((_GROUP,), jnp.int32),
            pltpu.VMEM((_GROUP,), jnp.int32),
            pltpu.VMEM((3 * _GROUP,), jnp.float32),
            pltpu.VMEM((3 * _GROUP,), jnp.int32),
            pltpu.VMEM((3 * _GROUP,), jnp.float32),
            pltpu.VMEM_SHARED((3 * _GROUP,), jnp.float32),
            pltpu.SemaphoreType.DMA,
            pltpu.SemaphoreType.DMA,
        ],
    )(functools.partial(_sc_body, row_off=_N_TC))
    sc_parts = sc_kernel(logits, labels_i32)
    tc_parts = _tc_bins(logits, labels_i32, _N_TC, c)

    p = sc_parts[0] + sc_parts[1]
    cnt = p[0:_N_BINS] + tc_parts[0, 0:_N_BINS]
    csum = p[_GROUP:_GROUP + _N_BINS] + tc_parts[1, 0:_N_BINS]
    asum = p[2 * _GROUP:2 * _GROUP + _N_BINS] + tc_parts[2, 0:_N_BINS]
    safe = jnp.maximum(cnt, 1.0)
    contrib = jnp.abs(csum / safe - asum / safe) * (cnt / n)
    ece = jnp.sum(jnp.where(cnt > 0, contrib, 0.0), dtype=jnp.float32)
    return ece.reshape(1)
